# baseline (speedup 1.0000x reference)
import functools

import jax
import jax.numpy as jnp
from jax import lax
from jax.experimental import pallas as pl
from jax.experimental.pallas import tpu as pltpu
from jax.experimental.pallas import tpu_sc as plsc

D = 64
NBUF = 4
BU = 128
PITCH = 129


@functools.cache
def _make_sc_gather(T: int, B0: int):
    n_workers = 32
    bw = B0 // n_workers
    upt = bw // BU
    n_units = T * upt
    n_rounds = n_units // NBUF
    mesh = plsc.VectorSubcoreMesh(core_axis_name="c", subcore_axis_name="s")

    @functools.partial(
        pl.kernel,
        mesh=mesh,
        compiler_params=pltpu.CompilerParams(needs_layout_passes=False),
        out_type=jax.ShapeDtypeStruct((T, D, B0), jnp.float32),
        scratch_types=[
            pltpu.VMEM((NBUF, BU), jnp.int32),
            pltpu.VMEM((NBUF, BU, 128), jnp.float32),
            pltpu.VMEM((D * PITCH,), jnp.float32),
            pltpu.VMEM((NBUF, D, BU), jnp.float32),
            pltpu.SemaphoreType.DMA((NBUF,)),
            pltpu.SemaphoreType.DMA((NBUF,)),
            pltpu.SemaphoreType.DMA((NBUF,)),
        ],
    )
    def k(wcat_hbm, xt_hbm, out_hbm, idx_v, gbuf, ibuf, tbuf, isem, gsem, osem):
        wid = lax.axis_index("s") * 2 + lax.axis_index("c")
        col0 = wid * bw

        iota = lax.iota(jnp.int32, 16)
        scat_base = [(iota + kk * 16) * PITCH for kk in range(4)]

        def unit_tb(u):
            return u // upt, col0 + (u % upt) * BU

        def idx_load(u, slot):
            t, b0 = unit_tb(u)
            return pltpu.make_async_copy(
                xt_hbm.at[t, pl.ds(b0, BU)], idx_v.at[slot], isem.at[slot]
            )

        def gather(slot):
            return pltpu.make_async_copy(
                wcat_hbm.at[idx_v.at[slot]], gbuf.at[slot], gsem.at[slot]
            )

        def write(u, slot):
            t, b0 = unit_tb(u)
            return pltpu.make_async_copy(
                tbuf.at[slot], out_hbm.at[t, :, pl.ds(b0, BU)], osem.at[slot]
            )

        def transpose(slot):
            def abody(i, carry):
                for j in range(2):
                    b = i * 2 + j
                    for kk in range(4):
                        v = gbuf[slot, b, pl.ds(kk * 16, 16)]
                        plsc.store_scatter(ibuf, [scat_base[kk] + b], v)
                return carry

            lax.fori_loop(0, BU // 2, abody, 0)

            def bbody(d, carry):
                for bg in range(8):
                    v = ibuf[pl.ds(d * PITCH + bg * 16, 16)]
                    tbuf[slot, d, pl.ds(bg * 16, 16)] = v
                return carry

            lax.fori_loop(0, D, bbody, 0)

        for s in range(NBUF):
            idx_load(s, s).start()
            idx_load(s, s).wait()
            gather(s).start()

        def round_body(r, carry):
            for slot in range(NBUF):
                u = r * NBUF + slot
                gather(slot).wait()
                nxt = u + NBUF

                @pl.when(nxt < n_units)
                def _():
                    idx_load(nxt, slot).start()

                @pl.when(u >= NBUF)
                def _():
                    write(u - NBUF, slot).wait()

                transpose(slot)
                write(u, slot).start()

                @pl.when(nxt < n_units)
                def _():
                    idx_load(nxt, slot).wait()
                    gather(slot).start()

            return carry

        lax.fori_loop(0, n_rounds, round_body, 0)

        for s in range(NBUF):
            write(n_units - NBUF + s, s).wait()

    return k


def kernel(x, W):
    B0, T = x.shape
    wcat = jnp.concatenate([W, W], axis=1)
    xt = x.T.astype(jnp.int32)
    k = _make_sc_gather(T, B0)
    out = k(wcat, xt)
    return jnp.transpose(out, (2, 0, 1))
